# baseline (speedup 1.0000x reference)
import functools

import jax
import jax.numpy as jnp
from jax import lax
from jax.experimental import pallas as pl
from jax.experimental.pallas import tpu as pltpu
from jax.experimental.pallas import tpu_sc as plsc

_PAD = 84
_VOCAB = 86
_VP = 128
_VZ = 96
_T = 20
_TP = 24
_L = 16
_NC, _NS = 2, 16
_NW = _NC * _NS


def _sc_histogram_packed(tokens_t):
  B = tokens_t.shape[1]
  rows = B // 2
  rpw = rows // _NW
  mesh = plsc.VectorSubcoreMesh(core_axis_name="c", subcore_axis_name="s")

  @functools.partial(
      pl.kernel,
      out_type=jax.ShapeDtypeStruct((rows * _VP,), jnp.int32),
      mesh=mesh,
      scratch_types=[
          pltpu.VMEM((_TP, 2 * rpw), jnp.int32),
          pltpu.VMEM((rpw * _VP,), jnp.int32),
          pltpu.SemaphoreType.DMA,
          pltpu.SemaphoreType.DMA,
      ],
      compiler_params=pltpu.CompilerParams(needs_layout_passes=False),
  )
  def hist_kernel(tok_hbm, out_hbm, tok_v, cnt_v, tsem, osem):
    wid = lax.axis_index("s") * _NC + lax.axis_index("c")
    base_r = wid * rpw

    tok_dmas = [
        pltpu.make_async_copy(
            tok_hbm.at[:, pl.ds(h * rows + base_r, rpw)],
            tok_v.at[:, pl.ds(h * rpw, rpw)], tsem)
        for h in range(2)
    ]
    for dma in tok_dmas:
      dma.start()

    zeros = jnp.zeros((_L,), jnp.int32)

    def zero_body(i, _):
      for c in range(_VZ // _L):
        cnt_v[pl.ds((i * (_VP // _L) + c) * _L, _L)] = zeros
      return 0

    lax.fori_loop(0, rpw, zero_body, 0, unroll=4)

    for dma in tok_dmas:
      dma.wait()

    lane = lax.iota(jnp.int32, _L)
    incr = [jnp.full((_L,), 1 << (16 * h), jnp.int32) for h in range(2)]

    def group_body(g, _):
      r0 = g * _L
      rows_v = (r0 + lane) * _VP

      def t_body(t, _):
        for h in range(2):
          tok = tok_v[t, pl.ds(h * rpw + r0, _L)]
          plsc.addupdate_scatter(cnt_v, [rows_v + tok], incr[h])
        return 0

      lax.fori_loop(0, _T, t_body, 0, unroll=4)
      return 0

    lax.fori_loop(0, rpw // _L, group_body, 0)

    out = pltpu.make_async_copy(
        cnt_v, out_hbm.at[pl.ds(base_r * _VP, rpw * _VP)], osem)
    out.start()
    out.wait()

  return hist_kernel(tokens_t)


def _tc_pool_mlp(packed, tw1, keep, b1, W2, b2, block_b):
  rows, _ = packed.shape
  B = rows * 2
  d = W2.shape[0]
  nblk = rows // block_b
  grid = (nblk, 2)

  def body(cnt_ref, tw1_ref, keep_ref, b1_ref, w2_ref, b2_ref, out_ref):
    h = pl.program_id(1)
    blk = cnt_ref[...]
    cnt = (jnp.right_shift(blk, 16 * h) & 0xFFFF).astype(jnp.float32)
    cntm = cnt * keep_ref[...]
    denom = jnp.maximum(jnp.sum(cntm, axis=1, keepdims=True), 1.0)
    hid = jnp.dot(cntm, tw1_ref[...],
                  preferred_element_type=jnp.float32) / denom + b1_ref[...]
    hid = 0.5 * hid * (1.0 + lax.erf(hid * 0.7071067811865476))
    out_ref[...] = jnp.dot(hid, w2_ref[...],
                           preferred_element_type=jnp.float32) + b2_ref[...]

  return pl.pallas_call(
      body,
      grid=grid,
      in_specs=[
          pl.BlockSpec((block_b, _VP), lambda i, h: (i, 0)),
          pl.BlockSpec((_VP, d), lambda i, h: (0, 0)),
          pl.BlockSpec((1, _VP), lambda i, h: (0, 0)),
          pl.BlockSpec((1, d), lambda i, h: (0, 0)),
          pl.BlockSpec((d, d), lambda i, h: (0, 0)),
          pl.BlockSpec((1, d), lambda i, h: (0, 0)),
      ],
      out_specs=pl.BlockSpec((block_b, d), lambda i, h: (i + h * nblk, 0)),
      out_shape=jax.ShapeDtypeStruct((B, d), jnp.float32),
  )(packed, tw1, keep, b1, W2, b2)


def kernel(tokens, table, W1, b1, W2, b2):
  B = tokens.shape[0]
  tokens_t = jnp.zeros((_TP, B), jnp.int32).at[:_T].set(tokens.T)
  packed = _sc_histogram_packed(tokens_t).reshape(B // 2, _VP)

  table_pad = jnp.zeros((_VP, table.shape[1]), table.dtype).at[:_VOCAB].set(table)
  tw1 = table_pad @ W1
  col = jnp.arange(_VP)
  keep = ((col < _VOCAB) & (col != _PAD)).astype(jnp.float32).reshape(1, _VP)

  return _tc_pool_mlp(packed, tw1, keep,
                      b1.reshape(1, -1), W2, b2.reshape(1, -1), block_b=8192)


# speedup vs baseline: 1.0207x; 1.0207x over previous
import functools

import jax
import jax.numpy as jnp
from jax import lax
from jax.experimental import pallas as pl
from jax.experimental.pallas import tpu as pltpu
from jax.experimental.pallas import tpu_sc as plsc

_PAD = 84
_VOCAB = 86
_VP = 128
_VZ = 96
_T = 20
_TP = 24
_L = 16
_NC, _NS = 2, 16
_NW = _NC * _NS


def _sc_histogram(tokens_t):
  B = tokens_t.shape[1]
  bpw = B // _NW
  mesh = plsc.VectorSubcoreMesh(core_axis_name="c", subcore_axis_name="s")

  @functools.partial(
      pl.kernel,
      out_type=jax.ShapeDtypeStruct((B * _VP,), jnp.float32),
      mesh=mesh,
      scratch_types=[
          pltpu.VMEM((_TP, bpw), jnp.int32),
          pltpu.VMEM((bpw * _VP,), jnp.float32),
          pltpu.SemaphoreType.DMA,
          pltpu.SemaphoreType.DMA,
      ],
      compiler_params=pltpu.CompilerParams(needs_layout_passes=False),
  )
  def hist_kernel(tok_hbm, out_hbm, tok_v, cnt_v, tsem, osem):
    wid = lax.axis_index("s") * _NC + lax.axis_index("c")
    base = wid * bpw
    hbpw = bpw // 2

    tok_dma = pltpu.make_async_copy(
        tok_hbm.at[:, pl.ds(base, bpw)], tok_v, tsem)
    tok_dma.start()

    zeros = jnp.zeros((_L,), jnp.float32)
    ones = jnp.ones((_L,), jnp.float32)
    lane = lax.iota(jnp.int32, _L)
    ngrp = hbpw // _L

    def zero_body(i, _):
      row = i * (_VP // _L)
      for c in range(_VZ // _L):
        cnt_v[pl.ds((row + c) * _L, _L)] = zeros
      return 0

    def make_group_body(s_half):
      def group_body(g, _):
        s0 = s_half + g * _L
        s1 = s_half + (g + ngrp // 2) * _L
        rows_a = (s0 + lane) * _VP
        rows_b = (s1 + lane) * _VP
        def t_body(t, _):
          tok_a = tok_v[t, pl.ds(s0, _L)]
          tok_b = tok_v[t, pl.ds(s1, _L)]
          plsc.addupdate_scatter(cnt_v, [rows_a + tok_a], ones)
          plsc.addupdate_scatter(cnt_v, [rows_b + tok_b], ones)
          return 0

        lax.fori_loop(0, _T, t_body, 0, unroll=4)
        return 0
      return group_body

    lax.fori_loop(0, hbpw, zero_body, 0, unroll=4)
    tok_dma.wait()
    lax.fori_loop(0, ngrp // 2, make_group_body(0), 0)
    out0 = pltpu.make_async_copy(
        cnt_v.at[pl.ds(0, hbpw * _VP)],
        out_hbm.at[pl.ds(base * _VP, hbpw * _VP)], osem)
    out0.start()

    def zero_body1(i, _):
      return zero_body(i + hbpw, _)

    lax.fori_loop(0, hbpw, zero_body1, 0, unroll=4)
    lax.fori_loop(0, ngrp // 2, make_group_body(hbpw), 0)
    out1 = pltpu.make_async_copy(
        cnt_v.at[pl.ds(hbpw * _VP, hbpw * _VP)],
        out_hbm.at[pl.ds((base + hbpw) * _VP, hbpw * _VP)], osem)
    out1.start()
    out0.wait()
    out1.wait()

  return hist_kernel(tokens_t)


def _tc_pool_mlp(counts, table_pad, W1, b1, W2, b2, block_b):
  B = counts.shape[0]
  grid = (B // block_b,)

  def body(cnt_ref, tw1_ref, b1_ref, w2_ref, b2_ref, out_ref):
    cnt = cnt_ref[...]
    col = lax.broadcasted_iota(jnp.int32, (1, _VP), 1)
    keep = jnp.logical_and(col != _PAD, col < _VOCAB)
    cntm = jnp.where(keep, cnt, 0.0)
    denom = jnp.maximum(jnp.sum(cntm, axis=1, keepdims=True), 1.0)
    h = jnp.dot(cntm, tw1_ref[...],
                preferred_element_type=jnp.float32) / denom + b1_ref[...]
    h = 0.5 * h * (1.0 + lax.erf(h * 0.7071067811865476))
    out_ref[...] = jnp.dot(h, w2_ref[...],
                           preferred_element_type=jnp.float32) + b2_ref[...]

  d = W2.shape[0]
  tw1 = table_pad @ W1
  return pl.pallas_call(
      body,
      grid=grid,
      in_specs=[
          pl.BlockSpec((block_b, _VP), lambda i: (i, 0)),
          pl.BlockSpec((_VP, d), lambda i: (0, 0)),
          pl.BlockSpec((1, d), lambda i: (0, 0)),
          pl.BlockSpec((d, d), lambda i: (0, 0)),
          pl.BlockSpec((1, d), lambda i: (0, 0)),
      ],
      out_specs=pl.BlockSpec((block_b, d), lambda i: (i, 0)),
      out_shape=jax.ShapeDtypeStruct((B, d), jnp.float32),
  )(counts, tw1, b1, W2, b2)


def kernel(tokens, table, W1, b1, W2, b2):
  B = tokens.shape[0]
  tokens_t = jnp.zeros((_TP, B), jnp.int32).at[:_T].set(tokens.T)
  table_pad = jnp.zeros((_VP, table.shape[1]), table.dtype).at[:_VOCAB].set(table)
  counts = _sc_histogram(tokens_t).reshape(B, _VP)
  return _tc_pool_mlp(counts, table_pad, W1,
                      b1.reshape(1, -1), W2, b2.reshape(1, -1), block_b=8192)
